# baseline (speedup 1.0000x reference)
import functools

import jax
import jax.numpy as jnp
from jax import lax
from jax.experimental import pallas as pl
from jax.experimental.pallas import tpu as pltpu
from jax.experimental.pallas import tpu_sc as plsc

NB = 32


def _sc_embed_add(xr, posf, tab):
    r2, d = xr.shape
    info = plsc.get_sparse_core_info()
    nc, ns = info.num_cores, info.num_subcores
    nw = nc * ns
    per_w = r2 // nw
    nblk = per_w // NB
    mesh = plsc.VectorSubcoreMesh(core_axis_name="c", subcore_axis_name="s")

    @functools.partial(
        pl.kernel,
        out_type=jax.ShapeDtypeStruct((r2, d), jnp.float32),
        mesh=mesh,
        scratch_types=[
            pltpu.VMEM((NB, d), jnp.float32),
            pltpu.VMEM((NB, d), jnp.float32),
            pltpu.VMEM((NB,), jnp.float32),
            pltpu.VMEM((NB,), jnp.int32),
            pltpu.SemaphoreType.DMA,
        ],
    )
    def body(x_hbm, posf_hbm, tab_hbm, out_hbm, xbuf, pebuf, idxf, idx, sem):
        wid = lax.axis_index("s") * nc + lax.axis_index("c")
        base = wid * per_w

        def blk(i, carry):
            b0 = base + i * NB
            pltpu.sync_copy(x_hbm.at[pl.ds(b0, NB)], xbuf)
            pltpu.sync_copy(posf_hbm.at[pl.ds(b0, NB)], idxf)
            for k in range(NB // 16):
                v = idxf[pl.ds(k * 16, 16)]
                idx[pl.ds(k * 16, 16)] = (v + 0.5).astype(jnp.int32)
            pltpu.async_copy(tab_hbm.at[idx], pebuf, sem).wait()

            def add_row(r, carry2):
                @plsc.parallel_loop(0, d // 16, unroll=8)
                def add_chunk(j):
                    sl = pl.ds(j * 16, 16)
                    xbuf[r, sl] = xbuf[r, sl] + pebuf[r, sl]

                return carry2

            lax.fori_loop(0, NB, add_row, 0)
            pltpu.sync_copy(xbuf, out_hbm.at[pl.ds(b0, NB)])
            return carry

        lax.fori_loop(0, nblk, blk, 0)

    return body(xr, posf, tab)


def kernel(x, pos, pos_embed):
    b, s1, e = x.shape
    half = e // 2
    nrows = pos_embed.shape[0]
    xr = x.reshape(b * s1 * 2, half)
    posf = jnp.pad(
        pos, ((0, 0), (1, 0), (0, 0)), constant_values=float(nrows)
    ).reshape(-1)
    tab = jnp.concatenate(
        [pos_embed, jnp.zeros((1, half), pos_embed.dtype)], axis=0
    )
    out = _sc_embed_add(xr, posf, tab)
    return out.reshape(b, s1, e)
